# baseline (speedup 1.0000x reference)
import jax
import jax.numpy as jnp
from jax import lax
from jax.experimental import pallas as pl
from jax.experimental.pallas import tpu as pltpu
from jax.experimental.pallas import tpu_sc as plsc

NT = 64
WIN = 16
LANES = 16

_B, _N, _D = 32, 1024, 768
_SC_B = 8
_NWORKERS = 32
_SC_NBUF = 2
_CH_OUT = 4
_CH_IN = _CH_OUT * WIN

_TC_NBUF = 16
_TC_ROWS = 32


def _make_sc_pool(sc_b):
    rows_out = sc_b * NT
    w_out = rows_out // _NWORKERS
    w_in = w_out * WIN
    nch = w_out // _CH_OUT
    ncol = _D // LANES

    def body(x_hbm, o_hbm, in_buf, out_buf, in_sems, out_sems):
        c = lax.axis_index("c")
        s = lax.axis_index("s")
        wid = s * 2 + c
        in_base = wid * w_in
        out_base = wid * w_out

        def start_in(g, slot):
            pltpu.make_async_copy(
                x_hbm.at[pl.ds(in_base + g * _CH_IN, _CH_IN)],
                in_buf.at[slot],
                in_sems.at[slot],
            ).start()

        def wait_in(slot):
            pltpu.make_async_copy(
                x_hbm.at[pl.ds(0, _CH_IN)],
                in_buf.at[slot],
                in_sems.at[slot],
            ).wait()

        def start_out(g, slot):
            pltpu.make_async_copy(
                out_buf.at[slot],
                o_hbm.at[pl.ds(out_base + g * _CH_OUT, _CH_OUT)],
                out_sems.at[slot],
            ).start()

        def wait_out(g, slot):
            pltpu.make_async_copy(
                out_buf.at[slot],
                o_hbm.at[pl.ds(out_base + g * _CH_OUT, _CH_OUT)],
                out_sems.at[slot],
            ).wait()

        for b in range(_SC_NBUF):
            start_in(b, b)

        def chunk_group(g0):
            for b in range(_SC_NBUF):
                g = g0 + b
                wait_in(b)

                @pl.when(g >= _SC_NBUF)
                def _():
                    wait_out(g - _SC_NBUF, b)

                for o in range(_CH_OUT):
                    def row_body(r, accs, o=o, b=b):
                        row = o * WIN + r
                        return tuple(
                            accs[j] + in_buf[b, row, pl.ds(j * LANES, LANES)]
                            for j in range(ncol)
                        )

                    accs0 = tuple(
                        in_buf[b, o * WIN, pl.ds(j * LANES, LANES)]
                        for j in range(ncol)
                    )
                    accs = lax.fori_loop(1, WIN, row_body, accs0)
                    for j in range(ncol):
                        out_buf[b, o, pl.ds(j * LANES, LANES)] = accs[j]

                start_out(g, b)

                @pl.when(g + _SC_NBUF < nch)
                def _():
                    start_in(g + _SC_NBUF, b)

        pl.loop(0, nch, step=_SC_NBUF)(chunk_group)

        for b in range(_SC_NBUF):
            wait_out(nch - _SC_NBUF + b, b)

    return pl.kernel(
        body,
        out_type=jax.ShapeDtypeStruct((rows_out, _D), jnp.float32),
        mesh=plsc.VectorSubcoreMesh(core_axis_name="c", subcore_axis_name="s"),
        scratch_types=[
            pltpu.VMEM((_SC_NBUF, _CH_IN, _D), jnp.float32),
            pltpu.VMEM((_SC_NBUF, _CH_OUT, _D), jnp.float32),
            pltpu.SemaphoreType.DMA((_SC_NBUF,)),
            pltpu.SemaphoreType.DMA((_SC_NBUF,)),
        ],
    )


def _tc_body(x_hbm, o_ref, buf, sems):
    i = pl.program_id(0)
    nsteps = pl.num_programs(0)
    base = _SC_B * NT

    def start(block, slot):
        pltpu.make_async_copy(
            x_hbm.at[pl.ds(base + block * _TC_ROWS, _TC_ROWS)],
            buf.at[slot],
            sems.at[slot],
        ).start()

    @pl.when(i == 0)
    def _():
        for k in range(_TC_NBUF):
            start(k, k)

    slot = lax.rem(i, _TC_NBUF)
    pltpu.make_async_copy(
        x_hbm.at[pl.ds(0, _TC_ROWS)],
        buf.at[slot],
        sems.at[slot],
    ).wait()

    o_ref[...] = jnp.sum(buf[slot], axis=1) * (1.0 / WIN)

    @pl.when(i + _TC_NBUF < nsteps)
    def _():
        start(i + _TC_NBUF, slot)


def kernel(features):
    B, N, D = features.shape
    x3 = features.reshape(B * N // WIN, WIN, D)

    out_sc = _make_sc_pool(_SC_B)(features.reshape(B * N, D))

    tc_rows = (B - _SC_B) * NT
    nsteps = tc_rows // _TC_ROWS
    out_tc = pl.pallas_call(
        _tc_body,
        grid=(nsteps,),
        in_specs=[pl.BlockSpec(memory_space=pl.ANY)],
        out_specs=pl.BlockSpec(
            (_TC_ROWS, D), lambda b: (b + _SC_B * NT // _TC_ROWS, 0)
        ),
        out_shape=jax.ShapeDtypeStruct((B * NT, D), jnp.float32),
        scratch_shapes=[
            pltpu.VMEM((_TC_NBUF, _TC_ROWS, WIN, D), jnp.float32),
            pltpu.SemaphoreType.DMA((_TC_NBUF,)),
        ],
    )(x3)

    out = lax.dynamic_update_slice(out_tc, out_sc * (1.0 / WIN), (0, 0))
    return out.reshape(B, NT, D)


# speedup vs baseline: 1.0571x; 1.0571x over previous
import jax
import jax.numpy as jnp
from jax import lax
from jax.experimental import pallas as pl
from jax.experimental.pallas import tpu as pltpu

NT = 64
WIN = 16

_NBUF = 32
_ROWS = 16


def _pool_body(x_hbm, o_ref, buf, sems):
    i = pl.program_id(0)
    nsteps = pl.num_programs(0)

    def start(block, slot):
        pltpu.make_async_copy(
            x_hbm.at[pl.ds(block * _ROWS, _ROWS)],
            buf.at[slot],
            sems.at[slot],
        ).start()

    @pl.when(i == 0)
    def _():
        for k in range(_NBUF):
            start(k, k)

    slot = lax.rem(i, _NBUF)
    pltpu.make_async_copy(
        x_hbm.at[pl.ds(0, _ROWS)],
        buf.at[slot],
        sems.at[slot],
    ).wait()

    o_ref[...] = jnp.sum(buf[slot], axis=1) * (1.0 / WIN)

    @pl.when(i + _NBUF < nsteps)
    def _():
        start(i + _NBUF, slot)


def kernel(features):
    B, N, D = features.shape
    nblocks = B * N // (_ROWS * WIN)
    x = features.reshape(B * N // WIN, WIN, D)
    out = pl.pallas_call(
        _pool_body,
        grid=(nblocks,),
        in_specs=[pl.BlockSpec(memory_space=pl.ANY)],
        out_specs=pl.BlockSpec((_ROWS, D), lambda b: (b, 0)),
        out_shape=jax.ShapeDtypeStruct((B * NT, D), jnp.float32),
        scratch_shapes=[
            pltpu.VMEM((_NBUF, _ROWS, WIN, D), jnp.float32),
            pltpu.SemaphoreType.DMA((_NBUF,)),
        ],
    )(x)
    return out.reshape(B, NT, D)


# speedup vs baseline: 1.5833x; 1.4978x over previous
import jax
import jax.numpy as jnp
from jax import lax
from jax.experimental import pallas as pl
from jax.experimental.pallas import tpu as pltpu

NT = 64
WIN = 16

_NBUF = 32
_ROWS = 32


def _pool_body(x_hbm, o_ref, buf, sems):
    i = pl.program_id(0)
    nsteps = pl.num_programs(0)

    def start(block, slot):
        pltpu.make_async_copy(
            x_hbm.at[pl.ds(block * _ROWS, _ROWS)],
            buf.at[slot],
            sems.at[slot],
        ).start()

    @pl.when(i == 0)
    def _():
        for k in range(_NBUF):
            start(k, k)

    slot = lax.rem(i, _NBUF)
    pltpu.make_async_copy(
        x_hbm.at[pl.ds(0, _ROWS)],
        buf.at[slot],
        sems.at[slot],
    ).wait()

    o_ref[...] = jnp.sum(buf[slot], axis=1) * (1.0 / WIN)

    @pl.when(i + _NBUF < nsteps)
    def _():
        start(i + _NBUF, slot)


def kernel(features):
    B, N, D = features.shape
    nblocks = B * N // (_ROWS * WIN)
    x = features.reshape(B * N // WIN, WIN, D)
    out = pl.pallas_call(
        _pool_body,
        grid=(nblocks,),
        in_specs=[pl.BlockSpec(memory_space=pl.ANY)],
        out_specs=pl.BlockSpec((_ROWS, D), lambda b: (b, 0)),
        out_shape=jax.ShapeDtypeStruct((B * NT, D), jnp.float32),
        scratch_shapes=[
            pltpu.VMEM((_NBUF, _ROWS, WIN, D), jnp.float32),
            pltpu.SemaphoreType.DMA((_NBUF,)),
        ],
    )(x)
    return out.reshape(B, NT, D)
